# baseline (speedup 1.0000x reference)
import functools

import jax
import jax.numpy as jnp
from jax import lax
from jax.experimental import pallas as pl
from jax.experimental.pallas import tpu as pltpu
from jax.experimental.pallas import tpu_sc as plsc

_B = 16384
_W = 5
_D = 64
_H = 512
_O = 50

_N = _B * _W
_NC = 2
_NS = 16
_NW = _NC * _NS
_RPW = _N // _NW
_CHUNK = 640
_NCHUNKS = _RPW // _CHUNK


@functools.lru_cache(maxsize=1)
def _make_sc_gather():
    @functools.partial(
        pl.kernel,
        mesh=plsc.VectorSubcoreMesh(core_axis_name="c", subcore_axis_name="s"),
        out_type=jax.ShapeDtypeStruct((_N, _D), jnp.float32),
        scratch_types=[
            pltpu.VMEM((_RPW,), jnp.int32),
            pltpu.VMEM((_CHUNK, _D), jnp.float32),
            pltpu.SemaphoreType.DMA,
        ],
        compiler_params=pltpu.CompilerParams(use_tc_tiling_on_sc=False),
    )
    def _sc_gather(table_hbm, idx_hbm, out_hbm, idx_v, rows_v, sem):
        wid = lax.axis_index("s") * _NC + lax.axis_index("c")
        base = wid * _RPW
        pltpu.sync_copy(idx_hbm.at[pl.ds(base, _RPW)], idx_v)
        for c in range(_NCHUNKS):
            off = c * _CHUNK
            pltpu.async_copy(
                table_hbm.at[idx_v.at[pl.ds(off, _CHUNK)]], rows_v, sem
            ).wait()
            pltpu.sync_copy(rows_v, out_hbm.at[pl.ds(base + off, _CHUNK)])

    return _sc_gather


_BB = 1024


def _mlp_body(e_ref, w1_ref, b1_ref, w2_ref, b2_ref, out_ref):
    e = e_ref[...]
    h = jnp.tanh(
        jnp.dot(e, w1_ref[...], preferred_element_type=jnp.float32) + b1_ref[...]
    )
    o = jnp.dot(h, w2_ref[...], preferred_element_type=jnp.float32) + b2_ref[...]
    m = jnp.max(o, axis=1, keepdims=True)
    s = jnp.log(jnp.sum(jnp.exp(o - m), axis=1, keepdims=True))
    out_ref[...] = o - m - s


def _tc_mlp(e, W1, b1, W2, b2):
    grid = (_B // _BB,)
    return pl.pallas_call(
        _mlp_body,
        grid=grid,
        in_specs=[
            pl.BlockSpec((_BB, _W * _D), lambda i: (i, 0)),
            pl.BlockSpec((_W * _D, _H), lambda i: (0, 0)),
            pl.BlockSpec((1, _H), lambda i: (0, 0)),
            pl.BlockSpec((_H, _O), lambda i: (0, 0)),
            pl.BlockSpec((1, _O), lambda i: (0, 0)),
        ],
        out_specs=pl.BlockSpec((_BB, _O), lambda i: (i, 0)),
        out_shape=jax.ShapeDtypeStruct((_B, _O), jnp.float32),
    )(e, W1, b1, W2, b2)


def kernel(x, table, W1, b1, W2, b2):
    idx = x.reshape(_N)
    e = _make_sc_gather()(table, idx)
    e = e.reshape(_B, _W * _D)
    return _tc_mlp(e, W1, b1.reshape(1, _H), W2, b2.reshape(1, _O))


# speedup vs baseline: 2.3066x; 2.3066x over previous
import functools

import jax
import jax.numpy as jnp
from jax import lax
from jax.experimental import pallas as pl
from jax.experimental.pallas import tpu as pltpu
from jax.experimental.pallas import tpu_sc as plsc

_B = 16384
_W = 5
_D = 64
_H = 512
_O = 50
_V = 1000000

_N = _B * _W
_NC = 2
_NS = 16
_NW = _NC * _NS
_BPW = _B // _NW
_CB = 64
_CI = _CB * _W
_NCHUNKS = _BPW // _CB


@functools.lru_cache(maxsize=1)
def _make_sc_gather():
    @functools.partial(
        pl.kernel,
        mesh=plsc.VectorSubcoreMesh(core_axis_name="c", subcore_axis_name="s"),
        out_type=jax.ShapeDtypeStruct((_B, _W * _D), jnp.float32),
        scratch_types=[
            pltpu.SMEM((_CI,), jnp.int32),
            pltpu.VMEM((_CI,), jnp.int32),
            pltpu.VMEM((_CB, _W * _D), jnp.float32),
            pltpu.SemaphoreType.DMA,
        ],
    )
    def _sc_gather(table3_hbm, idx_hbm, out_hbm, idx_s, idx_v, compact, sem):
        wid = lax.axis_index("s") * _NC + lax.axis_index("c")
        brow = wid * _BPW

        def chunk_body(ci, carry):
            row0 = brow + ci * _CB
            pltpu.sync_copy(idx_hbm.at[pl.ds(row0 * _W, _CI)], idx_v)
            for k in range(_CI // 16):
                v16 = idx_v[pl.ds(k * 16, 16)]
                for l in range(16):
                    idx_s[k * 16 + l] = v16[l]

            def issue_body(b, carry2):
                for w in range(_W):
                    v = idx_s[b * _W + w]
                    t = lax.shift_right_logical(v, 3)
                    s = lax.bitwise_and(v, 7)
                    pltpu.async_copy(
                        table3_hbm.at[t, s],
                        compact.at[b, pl.ds(w * _D, _D)],
                        sem,
                    )
                return carry2

            lax.fori_loop(0, _CB, issue_body, 0)

            def drain_body(b, carry2):
                for w in range(_W):
                    pltpu.make_async_copy(
                        table3_hbm.at[0, 0],
                        compact.at[b, pl.ds(w * _D, _D)],
                        sem,
                    ).wait()
                return carry2

            lax.fori_loop(0, _CB, drain_body, 0)
            pltpu.sync_copy(compact, out_hbm.at[pl.ds(row0, _CB)])
            return carry

        lax.fori_loop(0, _NCHUNKS, chunk_body, 0)

    return _sc_gather


_BB = 1024


def _mlp_body(e_ref, w1_ref, b1_ref, w2_ref, b2_ref, out_ref):
    e = e_ref[...]
    h = jnp.tanh(
        jnp.dot(e, w1_ref[...], preferred_element_type=jnp.float32) + b1_ref[...]
    )
    o = jnp.dot(h, w2_ref[...], preferred_element_type=jnp.float32) + b2_ref[...]
    m = jnp.max(o, axis=1, keepdims=True)
    s = jnp.log(jnp.sum(jnp.exp(o - m), axis=1, keepdims=True))
    out_ref[...] = o - m - s


def _tc_mlp(e, W1, b1, W2, b2):
    grid = (_B // _BB,)
    return pl.pallas_call(
        _mlp_body,
        grid=grid,
        in_specs=[
            pl.BlockSpec((_BB, _W * _D), lambda i: (i, 0)),
            pl.BlockSpec((_W * _D, _H), lambda i: (0, 0)),
            pl.BlockSpec((1, _H), lambda i: (0, 0)),
            pl.BlockSpec((_H, _O), lambda i: (0, 0)),
            pl.BlockSpec((1, _O), lambda i: (0, 0)),
        ],
        out_specs=pl.BlockSpec((_BB, _O), lambda i: (i, 0)),
        out_shape=jax.ShapeDtypeStruct((_B, _O), jnp.float32),
    )(e, W1, b1, W2, b2)


def kernel(x, table, W1, b1, W2, b2):
    idx = x.reshape(_N)
    table3 = table.reshape(_V // 8, 8, _D)
    e = _make_sc_gather()(table3, idx)
    return _tc_mlp(e, W1, b1.reshape(1, _H), W2, b2.reshape(1, _O))
